# baseline (speedup 1.0000x reference)
import jax
import jax.numpy as jnp
from jax import lax
from jax.experimental import pallas as pl
from jax.experimental.pallas import tpu as pltpu
from jax.experimental.pallas import tpu_sc as plsc

N_NODES = 100000
N_EDGES = 3200000
D = 16
B = 128
K = 4
C = K * B
N_SUBCORES = 16
IDX_ROWS = N_EDGES // B
IR_PER_TILE = 1568
IR_LAST_TILE = IDX_ROWS - 15 * IR_PER_TILE
CHUNKS = IR_PER_TILE // K
CHUNKS_LAST = IR_LAST_TILE // K
NODES_PER_TILE = 6256
NODES_LAST_TILE = N_NODES - 15 * NODES_PER_TILE


def _body(emb_hbm, eidx_hbm, hout_hbm, hin_hbm,
          rows_a, rows_b, idx_a, idx_b, acc, sem):
    cid = lax.axis_index("c")
    sid = lax.axis_index("s")

    def _zero(i, _):
        rows_a[i] = jnp.zeros((D,), jnp.float32)
        return 0

    lax.fori_loop(0, C, _zero, 0)
    n0 = sid * NODES_PER_TILE
    for k in range(NODES_PER_TILE // C):
        pltpu.sync_copy(rows_a, acc.at[pl.ds(n0 + k * C, C)])
    _full = (NODES_PER_TILE // C) * C

    @pl.when(sid < 15)
    def _():
        pltpu.sync_copy(rows_a.at[pl.ds(0, NODES_PER_TILE - _full)],
                        acc.at[pl.ds(n0 + _full, NODES_PER_TILE - _full)])

    @pl.when(sid == 15)
    def _():
        pltpu.sync_copy(rows_a.at[pl.ds(0, NODES_LAST_TILE - _full)],
                        acc.at[pl.ds(n0 + _full, NODES_LAST_TILE - _full)])

    plsc.subcore_barrier()

    i_base = sid * IR_PER_TILE
    e_base = i_base * B
    n_chunks = jnp.where(sid < 15, CHUNKS, CHUNKS_LAST)
    n_pairs = n_chunks // 2

    def _load(ci, rows_v, idx_v):
        pltpu.sync_copy(emb_hbm.at[pl.ds(e_base + ci * C, C)], rows_v)
        pltpu.sync_copy(eidx_hbm.at[cid, pl.ds(i_base + ci * K, K)], idx_v)

    def _scatter(rows_v, idx_v):
        return [
            pltpu.async_copy(rows_v.at[pl.ds(j * B, B)],
                             acc.at[idx_v.at[j]], sem, add=True)
            for j in range(K)
        ]

    _load(0, rows_a, idx_a)

    def _pair(p, _):
        descs_a = _scatter(rows_a, idx_a)
        _load(2 * p + 1, rows_b, idx_b)
        for d in descs_a:
            d.wait()
        descs_b = _scatter(rows_b, idx_b)
        _load(jnp.minimum(2 * p + 2, n_chunks - 1), rows_a, idx_a)
        for d in descs_b:
            d.wait()
        return 0

    lax.fori_loop(0, n_pairs, _pair, 0)
    plsc.subcore_barrier()

    out_hbm_sel = [hout_hbm, hin_hbm]
    for core in (0, 1):
        @pl.when(cid == core)
        def _(out=out_hbm_sel[core]):
            @pl.when(sid < 15)
            def _():
                pltpu.sync_copy(acc.at[pl.ds(n0, NODES_PER_TILE)],
                                out.at[pl.ds(n0, NODES_PER_TILE)])

            @pl.when(sid == 15)
            def _():
                pltpu.sync_copy(acc.at[pl.ds(n0, NODES_LAST_TILE)],
                                out.at[pl.ds(n0, NODES_LAST_TILE)])


@jax.jit
def _segment_sums(edge_embedding, eidx3):
    mesh = plsc.VectorSubcoreMesh(core_axis_name="c", subcore_axis_name="s")
    f = pl.kernel(
        _body,
        out_type=[
            jax.ShapeDtypeStruct((N_NODES, D), jnp.float32),
            jax.ShapeDtypeStruct((N_NODES, D), jnp.float32),
        ],
        mesh=mesh,
        scratch_types=[
            pltpu.VMEM((C, D), jnp.float32),
            pltpu.VMEM((C, D), jnp.float32),
            pltpu.VMEM((K, B), jnp.int32),
            pltpu.VMEM((K, B), jnp.int32),
            pltpu.VMEM_SHARED((N_NODES, D), jnp.float32),
            pltpu.SemaphoreType.DMA,
        ],
        compiler_params=pltpu.CompilerParams(use_tc_tiling_on_sc=False),
    )
    return f(edge_embedding, eidx3)


def kernel(edge_embedding, edge_index, coords):
    eidx3 = edge_index.reshape(2, IDX_ROWS, B)
    h_out, h_in = _segment_sums(edge_embedding, eidx3)
    return jnp.concatenate([h_out, h_in, coords], axis=1)


# speedup vs baseline: 1.1179x; 1.1179x over previous
import jax
import jax.numpy as jnp
from jax import lax
from jax.experimental import pallas as pl
from jax.experimental.pallas import tpu as pltpu
from jax.experimental.pallas import tpu_sc as plsc

N_NODES = 100000
N_EDGES = 3200000
D = 16
B = 128
K = 8
C = K * B
N_SUBCORES = 16
IDX_ROWS = N_EDGES // B
IR_PER_TILE = 1568
IR_LAST_TILE = IDX_ROWS - 15 * IR_PER_TILE
CHUNKS = IR_PER_TILE // K
CHUNKS_LAST = IR_LAST_TILE // K
NODES_PER_TILE = 6256
NODES_LAST_TILE = N_NODES - 15 * NODES_PER_TILE


def _body(emb_hbm, eidx_hbm, hout_hbm, hin_hbm, rows_v, idx_v, acc, sem):
    cid = lax.axis_index("c")
    sid = lax.axis_index("s")

    def _zero(i, _):
        rows_v[i] = jnp.zeros((D,), jnp.float32)
        return 0

    lax.fori_loop(0, C, _zero, 0)
    n0 = sid * NODES_PER_TILE
    for k in range(NODES_PER_TILE // C):
        pltpu.sync_copy(rows_v, acc.at[pl.ds(n0 + k * C, C)])
    _full = (NODES_PER_TILE // C) * C

    @pl.when(sid < 15)
    def _():
        pltpu.sync_copy(rows_v.at[pl.ds(0, NODES_PER_TILE - _full)],
                        acc.at[pl.ds(n0 + _full, NODES_PER_TILE - _full)])

    @pl.when(sid == 15)
    def _():
        pltpu.sync_copy(rows_v.at[pl.ds(0, NODES_LAST_TILE - _full)],
                        acc.at[pl.ds(n0 + _full, NODES_LAST_TILE - _full)])

    plsc.subcore_barrier()

    i_base = sid * IR_PER_TILE
    e_base = i_base * B
    n_chunks = jnp.where(sid < 15, CHUNKS, CHUNKS_LAST)

    def _chunk(ci, _):
        ld_rows = pltpu.async_copy(emb_hbm.at[pl.ds(e_base + ci * C, C)],
                                   rows_v, sem)
        ld_idx = pltpu.async_copy(eidx_hbm.at[cid, pl.ds(i_base + ci * K, K)],
                                  idx_v, sem)
        ld_rows.wait()
        ld_idx.wait()
        descs = [
            pltpu.async_copy(rows_v.at[pl.ds(j * B, B)],
                             acc.at[idx_v.at[j]], sem, add=True)
            for j in range(K)
        ]
        for d in descs:
            d.wait()
        return 0

    lax.fori_loop(0, n_chunks, _chunk, 0)
    plsc.subcore_barrier()

    out_hbm_sel = [hout_hbm, hin_hbm]
    for core in (0, 1):
        @pl.when(cid == core)
        def _(out=out_hbm_sel[core]):
            @pl.when(sid < 15)
            def _():
                pltpu.sync_copy(acc.at[pl.ds(n0, NODES_PER_TILE)],
                                out.at[pl.ds(n0, NODES_PER_TILE)])

            @pl.when(sid == 15)
            def _():
                pltpu.sync_copy(acc.at[pl.ds(n0, NODES_LAST_TILE)],
                                out.at[pl.ds(n0, NODES_LAST_TILE)])


@jax.jit
def _segment_sums(edge_embedding, eidx3):
    mesh = plsc.VectorSubcoreMesh(core_axis_name="c", subcore_axis_name="s")
    f = pl.kernel(
        _body,
        out_type=[
            jax.ShapeDtypeStruct((N_NODES, D), jnp.float32),
            jax.ShapeDtypeStruct((N_NODES, D), jnp.float32),
        ],
        mesh=mesh,
        scratch_types=[
            pltpu.VMEM((C, D), jnp.float32),
            pltpu.VMEM((K, B), jnp.int32),
            pltpu.VMEM_SHARED((N_NODES, D), jnp.float32),
            pltpu.SemaphoreType.DMA,
        ],
        compiler_params=pltpu.CompilerParams(use_tc_tiling_on_sc=False),
    )
    return f(edge_embedding, eidx3)


def kernel(edge_embedding, edge_index, coords):
    eidx3 = edge_index.reshape(2, IDX_ROWS, B)
    h_out, h_in = _segment_sums(edge_embedding, eidx3)
    return jnp.concatenate([h_out, h_in, coords], axis=1)


# speedup vs baseline: 1.1523x; 1.0308x over previous
import jax
import jax.numpy as jnp
from jax import lax
from jax.experimental import pallas as pl
from jax.experimental.pallas import tpu as pltpu
from jax.experimental.pallas import tpu_sc as plsc

N_NODES = 100000
N_EDGES = 3200000
D = 16
B = 128
K = 14
C = K * B
N_SUBCORES = 16
IDX_ROWS = N_EDGES // B
IR_PER_TILE = 1568
IR_LAST_TILE = IDX_ROWS - 15 * IR_PER_TILE
CHUNKS = IR_PER_TILE // K
CHUNKS_LAST = IR_LAST_TILE // K
K_TAIL = IR_LAST_TILE - CHUNKS_LAST * K
NODES_PER_TILE = 6256
NODES_LAST_TILE = N_NODES - 15 * NODES_PER_TILE


def _body(emb_hbm, eidx_hbm, hout_hbm, hin_hbm, rows_v, idx_v, acc, sem):
    cid = lax.axis_index("c")
    sid = lax.axis_index("s")

    def _zero(i, _):
        rows_v[i] = jnp.zeros((D,), jnp.float32)
        return 0

    lax.fori_loop(0, C, _zero, 0)
    n0 = sid * NODES_PER_TILE
    for k in range(NODES_PER_TILE // C):
        pltpu.sync_copy(rows_v, acc.at[pl.ds(n0 + k * C, C)])
    _full = (NODES_PER_TILE // C) * C

    @pl.when(sid < 15)
    def _():
        pltpu.sync_copy(rows_v.at[pl.ds(0, NODES_PER_TILE - _full)],
                        acc.at[pl.ds(n0 + _full, NODES_PER_TILE - _full)])

    @pl.when(sid == 15)
    def _():
        pltpu.sync_copy(rows_v.at[pl.ds(0, NODES_LAST_TILE - _full)],
                        acc.at[pl.ds(n0 + _full, NODES_LAST_TILE - _full)])

    plsc.subcore_barrier()

    i_base = sid * IR_PER_TILE
    e_base = i_base * B
    n_chunks = jnp.where(sid < 15, CHUNKS, CHUNKS_LAST)

    def _chunk(ci, _):
        ld_rows = pltpu.async_copy(emb_hbm.at[pl.ds(e_base + ci * C, C)],
                                   rows_v, sem)
        ld_idx = pltpu.async_copy(eidx_hbm.at[cid, pl.ds(i_base + ci * K, K)],
                                  idx_v, sem)
        ld_rows.wait()
        ld_idx.wait()
        descs = [
            pltpu.async_copy(rows_v.at[pl.ds(j * B, B)],
                             acc.at[idx_v.at[j]], sem, add=True)
            for j in range(K)
        ]
        for d in descs:
            d.wait()
        return 0

    lax.fori_loop(0, n_chunks, _chunk, 0)

    @pl.when(sid == 15)
    def _():
        t0 = i_base + CHUNKS_LAST * K
        ld_rows = pltpu.async_copy(emb_hbm.at[pl.ds(t0 * B, K_TAIL * B)],
                                   rows_v.at[pl.ds(0, K_TAIL * B)], sem)
        ld_idx = pltpu.async_copy(eidx_hbm.at[cid, pl.ds(t0, K_TAIL)],
                                  idx_v.at[pl.ds(0, K_TAIL)], sem)
        ld_rows.wait()
        ld_idx.wait()
        descs = [
            pltpu.async_copy(rows_v.at[pl.ds(j * B, B)],
                             acc.at[idx_v.at[j]], sem, add=True)
            for j in range(K_TAIL)
        ]
        for d in descs:
            d.wait()

    plsc.subcore_barrier()

    out_hbm_sel = [hout_hbm, hin_hbm]
    for core in (0, 1):
        @pl.when(cid == core)
        def _(out=out_hbm_sel[core]):
            @pl.when(sid < 15)
            def _():
                pltpu.sync_copy(acc.at[pl.ds(n0, NODES_PER_TILE)],
                                out.at[pl.ds(n0, NODES_PER_TILE)])

            @pl.when(sid == 15)
            def _():
                pltpu.sync_copy(acc.at[pl.ds(n0, NODES_LAST_TILE)],
                                out.at[pl.ds(n0, NODES_LAST_TILE)])


@jax.jit
def _segment_sums(edge_embedding, eidx3):
    mesh = plsc.VectorSubcoreMesh(core_axis_name="c", subcore_axis_name="s")
    f = pl.kernel(
        _body,
        out_type=[
            jax.ShapeDtypeStruct((N_NODES, D), jnp.float32),
            jax.ShapeDtypeStruct((N_NODES, D), jnp.float32),
        ],
        mesh=mesh,
        scratch_types=[
            pltpu.VMEM((C, D), jnp.float32),
            pltpu.VMEM((K, B), jnp.int32),
            pltpu.VMEM_SHARED((N_NODES, D), jnp.float32),
            pltpu.SemaphoreType.DMA,
        ],
        compiler_params=pltpu.CompilerParams(use_tc_tiling_on_sc=False),
    )
    return f(edge_embedding, eidx3)


def kernel(edge_embedding, edge_index, coords):
    eidx3 = edge_index.reshape(2, IDX_ROWS, B)
    h_out, h_in = _segment_sums(edge_embedding, eidx3)
    return jnp.concatenate([h_out, h_in, coords], axis=1)
